# baseline (speedup 1.0000x reference)
import functools

import jax
import jax.numpy as jnp
from jax import lax
from jax.experimental import pallas as pl
from jax.experimental.pallas import tpu as pltpu
from jax.experimental.pallas import tpu_sc as plsc

N_WORD = 128
B = 4096
L = 50
LP = L + 2

NC = 2
NS = 16
NW = NC * NS
ROWS_PER_W = B // NW
K = 8
N_ITER = ROWS_PER_W // K
BUF_ROWS = K * LP
NCHUNK = 4
GCHUNK = BUF_ROWS // NCHUNK


def _sc_embed(table, idx_p):
    mesh = plsc.VectorSubcoreMesh(core_axis_name="c", subcore_axis_name="s")

    @functools.partial(
        pl.kernel,
        mesh=mesh,
        out_type=jax.ShapeDtypeStruct((B * LP, N_WORD), jnp.float32),
        scratch_types=[
            pltpu.VMEM((ROWS_PER_W * LP,), jnp.int32),
            pltpu.VMEM((BUF_ROWS, N_WORD), jnp.float32),
            pltpu.SemaphoreType.DMA,
            pltpu.SemaphoreType.DMA,
        ],
    )
    def k(table_hbm, idx_hbm, out_hbm, idx_v, buf, gsem, wsem):
        wid = lax.axis_index("c") * NS + lax.axis_index("s")
        row_base = wid * (ROWS_PER_W * LP)
        pltpu.sync_copy(idx_hbm.at[pl.ds(row_base, ROWS_PER_W * LP)], idx_v)

        zeros16 = jnp.zeros((16,), jnp.float32)

        def body(g, carry):
            base = g * BUF_ROWS
            gathers = [
                pltpu.async_copy(
                    table_hbm.at[idx_v.at[pl.ds(base + c * GCHUNK, GCHUNK)]],
                    buf.at[pl.ds(c * GCHUNK, GCHUNK)],
                    gsem,
                )
                for c in range(NCHUNK)
            ]
            for cp in gathers:
                cp.wait()
            for i in range(K):
                for r in (i * LP, i * LP + LP - 1):
                    for cc in range(N_WORD // 16):
                        buf[r, pl.ds(cc * 16, 16)] = zeros16
            pltpu.async_copy(
                buf, out_hbm.at[pl.ds(row_base + base, BUF_ROWS)], wsem
            ).wait()
            return carry

        lax.fori_loop(0, N_ITER, body, 0)

    return k(table, idx_p)


def kernel(table, indices):
    idx_p = jnp.pad(indices.astype(jnp.int32), ((0, 0), (1, 1)))
    out = _sc_embed(table, idx_p.reshape(-1))
    return out.reshape(B, LP, N_WORD)


# speedup vs baseline: 1.0005x; 1.0005x over previous
import functools

import jax
import jax.numpy as jnp
from jax import lax
from jax.experimental import pallas as pl
from jax.experimental.pallas import tpu as pltpu
from jax.experimental.pallas import tpu_sc as plsc

N_WORD = 128
B = 4096
L = 50
LP = L + 2

NC = 2
NS = 16
NW = NC * NS
ROWS_PER_W = B // NW
K = 8
N_ITER = ROWS_PER_W // K
BUF_ROWS = K * LP
NCHUNK = 4
GCHUNK = BUF_ROWS // NCHUNK


def _sc_embed(table, idx_p):
    mesh = plsc.VectorSubcoreMesh(core_axis_name="c", subcore_axis_name="s")

    @functools.partial(
        pl.kernel,
        mesh=mesh,
        out_type=jax.ShapeDtypeStruct((B * LP, N_WORD), jnp.float32),
        scratch_types=[
            pltpu.VMEM((ROWS_PER_W * LP,), jnp.int32),
            pltpu.VMEM((BUF_ROWS, N_WORD), jnp.float32),
            pltpu.VMEM((BUF_ROWS, N_WORD), jnp.float32),
            pltpu.SemaphoreType.DMA,
            pltpu.SemaphoreType.DMA,
            pltpu.SemaphoreType.DMA,
            pltpu.SemaphoreType.DMA,
        ],
    )
    def k(table_hbm, idx_hbm, out_hbm, idx_v, buf0, buf1, g0, g1, w0, w1):
        wid = lax.axis_index("c") * NS + lax.axis_index("s")
        row_base = wid * (ROWS_PER_W * LP)
        pltpu.sync_copy(idx_hbm.at[pl.ds(row_base, ROWS_PER_W * LP)], idx_v)

        zeros16 = jnp.zeros((16,), jnp.float32)
        bufs = (buf0, buf1)
        gsems = (g0, g1)
        wsems = (w0, w1)

        def fire_gathers(g, buf, sem):
            base = g * BUF_ROWS
            for c in range(NCHUNK):
                pltpu.async_copy(
                    table_hbm.at[idx_v.at[pl.ds(base + c * GCHUNK, GCHUNK)]],
                    buf.at[pl.ds(c * GCHUNK, GCHUNK)],
                    sem,
                )

        def wait_gathers(g, buf, sem):
            base = g * BUF_ROWS
            for c in range(NCHUNK):
                pltpu.make_async_copy(
                    table_hbm.at[idx_v.at[pl.ds(base + c * GCHUNK, GCHUNK)]],
                    buf.at[pl.ds(c * GCHUNK, GCHUNK)],
                    sem,
                ).wait()

        def zero_pad_rows(buf):
            for i in range(K):
                for r in (i * LP, i * LP + LP - 1):
                    for cc in range(N_WORD // 16):
                        buf[r, pl.ds(cc * 16, 16)] = zeros16

        def fire_write(g, buf, sem):
            pltpu.async_copy(
                buf, out_hbm.at[pl.ds(row_base + g * BUF_ROWS, BUF_ROWS)], sem
            )

        def wait_write(g, buf, sem):
            pltpu.make_async_copy(
                buf, out_hbm.at[pl.ds(row_base + g * BUF_ROWS, BUF_ROWS)], sem
            ).wait()

        fire_gathers(0, buf0, g0)
        fire_gathers(1, buf1, g1)

        def body(p, carry):
            g = 2 * p
            for b in range(2):
                wait_gathers(g + b, bufs[b], gsems[b])
                zero_pad_rows(bufs[b])
                fire_write(g + b, bufs[b], wsems[b])
            for b in range(2):
                wait_write(g + b, bufs[b], wsems[b])
                fire_gathers(g + 2 + b, bufs[b], gsems[b])
            return carry

        lax.fori_loop(0, N_ITER // 2 - 1, body, 0)

        g = N_ITER - 2
        for b in range(2):
            wait_gathers(g + b, bufs[b], gsems[b])
            zero_pad_rows(bufs[b])
            fire_write(g + b, bufs[b], wsems[b])
        for b in range(2):
            wait_write(g + b, bufs[b], wsems[b])

    return k(table, idx_p)


def kernel(table, indices):
    idx_p = jnp.pad(indices.astype(jnp.int32), ((0, 0), (1, 1)))
    out = _sc_embed(table, idx_p.reshape(-1))
    return out.reshape(B, LP, N_WORD)


# speedup vs baseline: 2.4668x; 2.4657x over previous
import functools

import jax
import jax.numpy as jnp
from jax import lax
from jax.experimental import pallas as pl
from jax.experimental.pallas import tpu as pltpu
from jax.experimental.pallas import tpu_sc as plsc

N_WORD = 128
B = 4096
L = 50
LP = L + 2

NC = 2
NS = 16
NW = NC * NS
ROWS_PER_W = B // NW
K = 8
N_ITER = ROWS_PER_W // K
BUF_ROWS = K * LP


def _sc_embed(table, indices):
    mesh = plsc.VectorSubcoreMesh(core_axis_name="c", subcore_axis_name="s")

    @functools.partial(
        pl.kernel,
        mesh=mesh,
        out_type=jax.ShapeDtypeStruct((B * LP, N_WORD), jnp.float32),
        scratch_types=[
            pltpu.VMEM((ROWS_PER_W, L), jnp.int32),
            pltpu.VMEM((BUF_ROWS, N_WORD), jnp.float32),
            pltpu.VMEM((BUF_ROWS, N_WORD), jnp.float32),
            pltpu.SemaphoreType.DMA,
            pltpu.SemaphoreType.DMA,
            pltpu.SemaphoreType.DMA,
            pltpu.SemaphoreType.DMA,
        ],
    )
    def k(table_hbm, idx_hbm, out_hbm, idx_v, buf0, buf1, g0, g1, w0, w1):
        wid = lax.axis_index("c") * NS + lax.axis_index("s")
        row_base = wid * (ROWS_PER_W * LP)
        pltpu.sync_copy(idx_hbm.at[pl.ds(wid * ROWS_PER_W, ROWS_PER_W)], idx_v)

        zeros16 = jnp.zeros((16,), jnp.float32)
        bufs = (buf0, buf1)
        gsems = (g0, g1)
        wsems = (w0, w1)

        for buf in bufs:
            for i in range(K):
                for r in (i * LP, i * LP + LP - 1):
                    for cc in range(N_WORD // 16):
                        buf[r, pl.ds(cc * 16, 16)] = zeros16

        def fire_gathers(g, buf, sem):
            for i in range(K):
                pltpu.async_copy(
                    table_hbm.at[idx_v.at[g * K + i]],
                    buf.at[pl.ds(i * LP + 1, L)],
                    sem,
                )

        def wait_gathers(g, buf, sem):
            for i in range(K):
                pltpu.make_async_copy(
                    table_hbm.at[idx_v.at[g * K + i]],
                    buf.at[pl.ds(i * LP + 1, L)],
                    sem,
                ).wait()

        def fire_write(g, buf, sem):
            pltpu.async_copy(
                buf, out_hbm.at[pl.ds(row_base + g * BUF_ROWS, BUF_ROWS)], sem
            )

        def wait_write(g, buf, sem):
            pltpu.make_async_copy(
                buf, out_hbm.at[pl.ds(row_base + g * BUF_ROWS, BUF_ROWS)], sem
            ).wait()

        fire_gathers(0, buf0, g0)
        fire_gathers(1, buf1, g1)

        def body(p, carry):
            g = 2 * p
            for b in range(2):
                wait_gathers(g + b, bufs[b], gsems[b])
                fire_write(g + b, bufs[b], wsems[b])
            for b in range(2):
                wait_write(g + b, bufs[b], wsems[b])
                fire_gathers(g + 2 + b, bufs[b], gsems[b])
            return carry

        lax.fori_loop(0, N_ITER // 2 - 1, body, 0)

        g = N_ITER - 2
        for b in range(2):
            wait_gathers(g + b, bufs[b], gsems[b])
            fire_write(g + b, bufs[b], wsems[b])
        for b in range(2):
            wait_write(g + b, bufs[b], wsems[b])

    return k(table, indices)


def kernel(table, indices):
    out = _sc_embed(table, indices.astype(jnp.int32))
    return out.reshape(B, LP, N_WORD)


# speedup vs baseline: 4.1349x; 1.6762x over previous
import functools

import jax
import jax.numpy as jnp
from jax import lax
from jax.experimental import pallas as pl
from jax.experimental.pallas import tpu as pltpu
from jax.experimental.pallas import tpu_sc as plsc

N_WORD = 128
B = 4096
L = 50
LP = L + 2

NC = 2
NS = 16
NW = NC * NS
ROWS_PER_W = B // NW
K = 8
N_ITER = ROWS_PER_W // K


def _sc_embed(table, indices):
    mesh = plsc.VectorSubcoreMesh(core_axis_name="c", subcore_axis_name="s")

    @functools.partial(
        pl.kernel,
        mesh=mesh,
        out_type=jax.ShapeDtypeStruct((B, LP, N_WORD), jnp.float32),
        scratch_types=[
            pltpu.VMEM((ROWS_PER_W, L), jnp.int32),
            pltpu.VMEM((K, LP, N_WORD), jnp.float32),
            pltpu.VMEM((K, LP, N_WORD), jnp.float32),
            pltpu.SemaphoreType.DMA,
            pltpu.SemaphoreType.DMA,
            pltpu.SemaphoreType.DMA,
            pltpu.SemaphoreType.DMA,
        ],
    )
    def k(table_hbm, idx_hbm, out_hbm, idx_v, buf0, buf1, g0, g1, w0, w1):
        wid = lax.axis_index("c") * NS + lax.axis_index("s")
        batch_base = wid * ROWS_PER_W
        pltpu.sync_copy(idx_hbm.at[pl.ds(batch_base, ROWS_PER_W)], idx_v)

        zeros16 = jnp.zeros((16,), jnp.float32)
        bufs = (buf0, buf1)
        gsems = (g0, g1)
        wsems = (w0, w1)

        for buf in bufs:
            for i in range(K):
                for r in (0, LP - 1):
                    for cc in range(N_WORD // 16):
                        buf[i, r, pl.ds(cc * 16, 16)] = zeros16

        def fire_gathers(g, buf, sem):
            for i in range(K):
                pltpu.async_copy(
                    table_hbm.at[idx_v.at[g * K + i]],
                    buf.at[i, pl.ds(1, L)],
                    sem,
                )

        def wait_gathers(g, buf, sem):
            for i in range(K):
                pltpu.make_async_copy(
                    table_hbm.at[idx_v.at[g * K + i]],
                    buf.at[i, pl.ds(1, L)],
                    sem,
                ).wait()

        def fire_write(g, buf, sem):
            pltpu.async_copy(
                buf, out_hbm.at[pl.ds(batch_base + g * K, K)], sem
            )

        def wait_write(g, buf, sem):
            pltpu.make_async_copy(
                buf, out_hbm.at[pl.ds(batch_base + g * K, K)], sem
            ).wait()

        fire_gathers(0, buf0, g0)
        fire_gathers(1, buf1, g1)

        def body(p, carry):
            g = 2 * p
            for b in range(2):
                wait_gathers(g + b, bufs[b], gsems[b])
                fire_write(g + b, bufs[b], wsems[b])
            for b in range(2):
                wait_write(g + b, bufs[b], wsems[b])
                fire_gathers(g + 2 + b, bufs[b], gsems[b])
            return carry

        lax.fori_loop(0, N_ITER // 2 - 1, body, 0)

        g = N_ITER - 2
        for b in range(2):
            wait_gathers(g + b, bufs[b], gsems[b])
            fire_write(g + b, bufs[b], wsems[b])
        for b in range(2):
            wait_write(g + b, bufs[b], wsems[b])

    return k(table, indices)


def kernel(table, indices):
    return _sc_embed(table, indices.astype(jnp.int32))


# speedup vs baseline: 7.5863x; 1.8347x over previous
import functools

import jax
import jax.numpy as jnp
from jax import lax
from jax.experimental import pallas as pl
from jax.experimental.pallas import tpu as pltpu
from jax.experimental.pallas import tpu_sc as plsc

N_WORD = 128
B = 4096
L = 50
LP = L + 2

NC = 2
NS = 16
NW = NC * NS
BW = B // NW
NBUF = 5


def _sc_embed(table, idx_t):
    mesh = plsc.VectorSubcoreMesh(core_axis_name="c", subcore_axis_name="s")

    @functools.partial(
        pl.kernel,
        mesh=mesh,
        out_type=jax.ShapeDtypeStruct((LP, B, N_WORD), jnp.float32),
        scratch_types=[
            pltpu.VMEM((L, BW), jnp.int32),
            pltpu.VMEM((BW, N_WORD), jnp.float32),
            pltpu.VMEM((BW, N_WORD), jnp.float32),
            pltpu.VMEM((BW, N_WORD), jnp.float32),
            pltpu.VMEM((BW, N_WORD), jnp.float32),
            pltpu.VMEM((BW, N_WORD), jnp.float32),
            pltpu.VMEM((BW, N_WORD), jnp.float32),
            pltpu.SemaphoreType.DMA,
            pltpu.SemaphoreType.DMA,
            pltpu.SemaphoreType.DMA,
            pltpu.SemaphoreType.DMA,
            pltpu.SemaphoreType.DMA,
            pltpu.SemaphoreType.DMA,
            pltpu.SemaphoreType.DMA,
        ],
    )
    def k(table_hbm, idx_hbm, out_hbm,
          idx_v, zbuf, b0_, b1_, b2_, b3_, b4_,
          zsem, g0_, g1_, g2_, g3_, g4_, wsem):
        wid = lax.axis_index("c") * NS + lax.axis_index("s")
        b0 = wid * BW
        bufs = (b0_, b1_, b2_, b3_, b4_)
        gsems = (g0_, g1_, g2_, g3_, g4_)

        idx_cp = [
            pltpu.async_copy(
                idx_hbm.at[s, pl.ds(b0, BW)], idx_v.at[s], zsem
            )
            for s in range(L)
        ]

        zeros16 = jnp.zeros((16,), jnp.float32)

        def zfill(r, carry):
            for cc in range(N_WORD // 16):
                zbuf[r, pl.ds(cc * 16, 16)] = zeros16
            return carry

        lax.fori_loop(0, BW, zfill, 0)

        for cp in idx_cp:
            cp.wait()

        def fire_gather(i, buf, sem):
            pltpu.async_copy(table_hbm.at[idx_v.at[i]], buf, sem)

        def wait_gather(i, buf, sem):
            pltpu.make_async_copy(table_hbm.at[idx_v.at[i]], buf, sem).wait()

        def fire_write(i, buf, sem):
            pltpu.async_copy(buf, out_hbm.at[i + 1, pl.ds(b0, BW)], sem)

        def wait_write(i, buf, sem):
            pltpu.make_async_copy(
                buf, out_hbm.at[i + 1, pl.ds(b0, BW)], sem
            ).wait()

        for j in range(NBUF):
            fire_gather(j, bufs[j], gsems[j])

        zw0 = pltpu.async_copy(zbuf, out_hbm.at[0, pl.ds(b0, BW)], zsem)
        zw1 = pltpu.async_copy(zbuf, out_hbm.at[LP - 1, pl.ds(b0, BW)], zsem)

        def body(p, carry):
            i = p * NBUF
            for j in range(NBUF):
                wait_gather(i + j, bufs[j], gsems[j])
                fire_write(i + j, bufs[j], wsem)
            for j in range(NBUF):
                wait_write(i + j, bufs[j], wsem)
                fire_gather(i + NBUF + j, bufs[j], gsems[j])
            return carry

        lax.fori_loop(0, L // NBUF - 2, body, 0)

        i = L - 2 * NBUF
        for j in range(NBUF):
            wait_gather(i + j, bufs[j], gsems[j])
            fire_write(i + j, bufs[j], wsem)
        for j in range(NBUF):
            wait_write(i + j, bufs[j], wsem)
            fire_gather(i + NBUF + j, bufs[j], gsems[j])
        i = L - NBUF
        for j in range(NBUF):
            wait_gather(i + j, bufs[j], gsems[j])
            fire_write(i + j, bufs[j], wsem)
        for j in range(NBUF):
            wait_write(i + j, bufs[j], wsem)
        zw0.wait()
        zw1.wait()

    return k(table, idx_t)


def kernel(table, indices):
    out = _sc_embed(table, indices.astype(jnp.int32).T)
    return out.transpose(1, 0, 2)


# speedup vs baseline: 7.6324x; 1.0061x over previous
import functools

import jax
import jax.numpy as jnp
from jax import lax
from jax.experimental import pallas as pl
from jax.experimental.pallas import tpu as pltpu
from jax.experimental.pallas import tpu_sc as plsc

N_WORD = 128
B = 4096
L = 50
LP = L + 2

NC = 2
NS = 16
NW = NC * NS
BW = B // NW
NBUF = 5


def _sc_embed(table, idx_t):
    mesh = plsc.VectorSubcoreMesh(core_axis_name="c", subcore_axis_name="s")

    @functools.partial(
        pl.kernel,
        mesh=mesh,
        out_type=jax.ShapeDtypeStruct((LP, B, N_WORD), jnp.float32),
        scratch_types=[
            pltpu.VMEM((L, BW), jnp.int32),
            pltpu.VMEM((BW, N_WORD), jnp.float32),
            pltpu.VMEM((BW, N_WORD), jnp.float32),
            pltpu.VMEM((BW, N_WORD), jnp.float32),
            pltpu.VMEM((BW, N_WORD), jnp.float32),
            pltpu.VMEM((BW, N_WORD), jnp.float32),
            pltpu.VMEM((BW, N_WORD), jnp.float32),
            pltpu.SemaphoreType.DMA,
            pltpu.SemaphoreType.DMA,
            pltpu.SemaphoreType.DMA,
            pltpu.SemaphoreType.DMA,
            pltpu.SemaphoreType.DMA,
            pltpu.SemaphoreType.DMA,
            pltpu.SemaphoreType.DMA,
        ],
    )
    def k(table_hbm, idx_hbm, out_hbm,
          idx_v, zbuf, b0_, b1_, b2_, b3_, b4_,
          zsem, g0_, g1_, g2_, g3_, g4_, wsem):
        wid = lax.axis_index("c") * NS + lax.axis_index("s")
        b0 = wid * BW
        bufs = (b0_, b1_, b2_, b3_, b4_)
        gsems = (g0_, g1_, g2_, g3_, g4_)

        idx_cp = [
            pltpu.async_copy(idx_hbm.at[:, pl.ds(b0, BW)], idx_v, zsem)
        ]

        zeros16 = jnp.zeros((16,), jnp.float32)

        def zfill(r, carry):
            for cc in range(N_WORD // 16):
                zbuf[r, pl.ds(cc * 16, 16)] = zeros16
            return carry

        lax.fori_loop(0, BW, zfill, 0)

        for cp in idx_cp:
            cp.wait()

        def fire_gather(i, buf, sem):
            pltpu.async_copy(table_hbm.at[idx_v.at[i]], buf, sem)

        def wait_gather(i, buf, sem):
            pltpu.make_async_copy(table_hbm.at[idx_v.at[i]], buf, sem).wait()

        def fire_write(i, buf, sem):
            pltpu.async_copy(buf, out_hbm.at[i + 1, pl.ds(b0, BW)], sem)

        def wait_write(i, buf, sem):
            pltpu.make_async_copy(
                buf, out_hbm.at[i + 1, pl.ds(b0, BW)], sem
            ).wait()

        for j in range(NBUF):
            fire_gather(j, bufs[j], gsems[j])

        zw0 = pltpu.async_copy(zbuf, out_hbm.at[0, pl.ds(b0, BW)], zsem)
        zw1 = pltpu.async_copy(zbuf, out_hbm.at[LP - 1, pl.ds(b0, BW)], zsem)

        def body(p, carry):
            i = p * NBUF
            for j in range(NBUF):
                wait_gather(i + j, bufs[j], gsems[j])
                fire_write(i + j, bufs[j], wsem)
            for j in range(NBUF):
                wait_write(i + j, bufs[j], wsem)
                fire_gather(i + NBUF + j, bufs[j], gsems[j])
            return carry

        lax.fori_loop(0, L // NBUF - 2, body, 0)

        i = L - 2 * NBUF
        for j in range(NBUF):
            wait_gather(i + j, bufs[j], gsems[j])
            fire_write(i + j, bufs[j], wsem)
        for j in range(NBUF):
            wait_write(i + j, bufs[j], wsem)
            fire_gather(i + NBUF + j, bufs[j], gsems[j])
        i = L - NBUF
        for j in range(NBUF):
            wait_gather(i + j, bufs[j], gsems[j])
            fire_write(i + j, bufs[j], wsem)
        for j in range(NBUF):
            wait_write(i + j, bufs[j], wsem)
        zw0.wait()
        zw1.wait()

    return k(table, idx_t)


def kernel(table, indices):
    out = _sc_embed(table, indices.astype(jnp.int32).T)
    return out.transpose(1, 0, 2)
